# baseline (speedup 1.0000x reference)
import functools

import jax
import jax.numpy as jnp
from jax import lax
from jax.experimental import pallas as pl
from jax.experimental.pallas import tpu as pltpu
from jax.experimental.pallas import tpu_sc as plsc

B = 16384
K = 32
KP = 8
NPH = K // KP
N_ROWS = 1000000
NC = 2
NS = 16
NW = NC * NS
BPW = B // NW
C = 16
NCH = BPW // C
L = 16


_mesh = plsc.VectorSubcoreMesh(core_axis_name="c", subcore_axis_name="s")


@functools.partial(
    pl.kernel,
    mesh=_mesh,
    compiler_params=pltpu.CompilerParams(needs_layout_passes=False),
    out_type=jax.ShapeDtypeStruct((B,), jnp.float32),
    scratch_types=[
        pltpu.VMEM((BPW,), jnp.int32),
        pltpu.VMEM((BPW,), jnp.int32),
        pltpu.VMEM((C, KP, 128), jnp.float32),
        pltpu.VMEM((C, KP, 128), jnp.float32),
        pltpu.VMEM((C, KP, 128), jnp.float32),
        pltpu.VMEM((C, KP, 128), jnp.float32),
        pltpu.VMEM((BPW,), jnp.float32),
        pltpu.SemaphoreType.DMA,
        pltpu.SemaphoreType.DMA,
    ],
)
def _pmf_sc(user_hbm, item_hbm, wu_t_hbm, wi_t_hbm, out_hbm,
            uvec, ivec, ub0, ub1, ib0, ib1, oacc, sem0, sem1):
    wid = lax.axis_index("s") * NC + lax.axis_index("c")
    base = wid * BPW

    pltpu.sync_copy(user_hbm.at[pl.ds(base, BPW)], uvec)
    pltpu.sync_copy(item_hbm.at[pl.ds(base, BPW)], ivec)

    bufs = [(ub0, ib0, sem0), (ub1, ib1, sem1)]

    def chunk(c, carry):
        uv = uvec[pl.ds(c * C, L)]
        iv = ivec[pl.ds(c * C, L)]
        ulane = jnp.bitwise_and(uv, 127)
        ilane = jnp.bitwise_and(iv, 127)
        ubase = lax.shift_left(lax.shift_right_logical(uv, 7), 7)
        ibase = lax.shift_left(lax.shift_right_logical(iv, 7), 7)
        sel = lax.iota(jnp.int32, L)
        zero = jnp.zeros((L,), jnp.int32)

        ubs = [pl.multiple_of(jnp.sum(jnp.where(sel == i, ubase, zero)), 128)
               for i in range(C)]
        ibs = [pl.multiple_of(jnp.sum(jnp.where(sel == i, ibase, zero)), 128)
               for i in range(C)]

        def fire(p, bu, bi, sem):
            ksl = pl.ds(p * KP, KP)
            for i in range(C):
                pltpu.async_copy(
                    wu_t_hbm.at[ksl, pl.ds(ubs[i], 128)], bu.at[i], sem)
                pltpu.async_copy(
                    wi_t_hbm.at[ksl, pl.ds(ibs[i], 128)], bi.at[i], sem)

        def drain(bu, bi, sem):
            for i in range(C):
                pltpu.make_async_copy(
                    wu_t_hbm.at[pl.ds(0, KP), pl.ds(0, 128)],
                    bu.at[i], sem).wait()
                pltpu.make_async_copy(
                    wi_t_hbm.at[pl.ds(0, KP), pl.ds(0, 128)],
                    bi.at[i], sem).wait()

        def accum(bu, bi, acc):
            for kk in range(KP):
                kvec = jnp.full((L,), kk, jnp.int32)
                u = plsc.load_gather(bu, [sel, kvec, ulane])
                v = plsc.load_gather(bi, [sel, kvec, ilane])
                acc = acc + u * v
            return acc

        acc = jnp.zeros((L,), jnp.float32)
        fire(0, *bufs[0])
        for p in range(NPH):
            if p + 1 < NPH:
                fire(p + 1, *bufs[(p + 1) % 2])
            drain(*bufs[p % 2])
            acc = accum(bufs[p % 2][0], bufs[p % 2][1], acc)

        oacc[pl.ds(c * C, L)] = acc
        return carry

    lax.fori_loop(0, NCH, chunk, 0)

    pltpu.sync_copy(oacc, out_hbm.at[pl.ds(base, BPW)])


def kernel(user, item, W_user, W_item):
    return _pmf_sc(user, item, W_user.T, W_item.T)


# speedup vs baseline: 1.0168x; 1.0168x over previous
import functools

import jax
import jax.numpy as jnp
from jax import lax
from jax.experimental import pallas as pl
from jax.experimental.pallas import tpu as pltpu
from jax.experimental.pallas import tpu_sc as plsc

B = 16384
K = 32
N_ROWS = 1000000
NC = 2
NS = 16
NW = NC * NS
BPW = B // NW
C = 16
NCH = BPW // C
L = 16


_mesh = plsc.VectorSubcoreMesh(core_axis_name="c", subcore_axis_name="s")


@functools.partial(
    pl.kernel,
    mesh=_mesh,
    compiler_params=pltpu.CompilerParams(needs_layout_passes=False),
    out_type=jax.ShapeDtypeStruct((B,), jnp.float32),
    scratch_types=[
        pltpu.VMEM((BPW,), jnp.int32),
        pltpu.VMEM((BPW,), jnp.int32),
        pltpu.VMEM((C, K, 128), jnp.float32),
        pltpu.VMEM((K, L), jnp.float32),
        pltpu.VMEM((K, L), jnp.float32),
        pltpu.VMEM((BPW,), jnp.float32),
        pltpu.SemaphoreType.DMA,
    ],
)
def _pmf_sc(user_hbm, item_hbm, wu_t_hbm, wi_t_hbm, out_hbm,
            uvec, ivec, slab, uex, iex, oacc, sem):
    wid = lax.axis_index("s") * NC + lax.axis_index("c")
    base = wid * BPW

    pltpu.sync_copy(user_hbm.at[pl.ds(base, BPW)], uvec)
    pltpu.sync_copy(item_hbm.at[pl.ds(base, BPW)], ivec)

    sel = lax.iota(jnp.int32, L)
    zero = jnp.zeros((L,), jnp.int32)

    def chunk(c, carry):
        def fetch_extract(vec_ref, tbl_hbm, ex_ref):
            v = vec_ref[pl.ds(c * C, L)]
            lane = jnp.bitwise_and(v, 127)
            vbase = lax.shift_left(lax.shift_right_logical(v, 7), 7)
            copies = []
            for i in range(C):
                off = pl.multiple_of(
                    jnp.sum(jnp.where(sel == i, vbase, zero)), 128)
                copies.append(pltpu.async_copy(
                    tbl_hbm.at[:, pl.ds(off, 128)], slab.at[i], sem))
            for cp in copies:
                cp.wait()
            for k in range(K):
                kvec = jnp.full((L,), k, jnp.int32)
                ex_ref[k, :] = plsc.load_gather(slab, [sel, kvec, lane])

        fetch_extract(uvec, wu_t_hbm, uex)
        fetch_extract(ivec, wi_t_hbm, iex)

        acc = jnp.zeros((L,), jnp.float32)
        for k in range(K):
            acc = acc + uex[k, :] * iex[k, :]
        oacc[pl.ds(c * C, L)] = acc
        return carry

    lax.fori_loop(0, NCH, chunk, 0)

    pltpu.sync_copy(oacc, out_hbm.at[pl.ds(base, BPW)])


def kernel(user, item, W_user, W_item):
    return _pmf_sc(user, item, W_user.T, W_item.T)
